# baseline (speedup 1.0000x reference)
import functools

import jax
import jax.numpy as jnp
from jax import lax
from jax.experimental import pallas as pl
from jax.experimental.pallas import tpu as pltpu
from jax.experimental.pallas import tpu_sc as plsc

_NUM_REQS = 128
_LANES = 16
_NUM_WORKERS = _NUM_REQS // _LANES


def _last_loc_sc(table_flat, pool_idx, prefix_lens, stride, num_tokens):
    info = plsc.get_sparse_core_info()
    num_cores = info.num_cores
    mesh = plsc.VectorSubcoreMesh(core_axis_name="c", subcore_axis_name="s")

    @functools.partial(
        pl.kernel,
        mesh=mesh,
        out_type=jax.ShapeDtypeStruct((_NUM_REQS,), jnp.int32),
        scratch_types=[
            pltpu.VMEM((_LANES,), jnp.int32),
            pltpu.VMEM((_LANES,), jnp.int32),
            pltpu.VMEM((_LANES,), jnp.int32),
            pltpu.VMEM((_LANES,), jnp.int32),
            pltpu.SemaphoreType.DMA,
        ],
    )
    def body(table_hbm, pool_hbm, len_hbm, out_hbm, pool_v, len_v, idx_v, res_v, sem):
        wid = lax.axis_index("s") * num_cores + lax.axis_index("c")

        @pl.when(wid < _NUM_WORKERS)
        def _():
            base = wid * _LANES
            pltpu.sync_copy(pool_hbm.at[pl.ds(base, _LANES)], pool_v)
            pltpu.sync_copy(len_hbm.at[pl.ds(base, _LANES)], len_v)
            lens = len_v[...]
            tok = pool_v[...] * stride + (lens - 1)
            idx_v[...] = jnp.clip(tok, 0, num_tokens - 1)
            pltpu.async_copy(table_hbm.at[idx_v], res_v, sem).wait()
            res_v[...] = jnp.where(lens > 0, res_v[...], jnp.int32(-1))
            pltpu.sync_copy(res_v, out_hbm.at[pl.ds(base, _LANES)])

    return body(table_flat, pool_idx, prefix_lens)


def kernel(req_to_token, req_pool_indices_tensor, prefix_lens_tensor):
    stride = req_to_token.shape[1]
    num_tokens = req_to_token.shape[0] * stride
    flat = req_to_token.reshape(-1).astype(jnp.int32)
    pool = req_pool_indices_tensor.astype(jnp.int32)
    lens = prefix_lens_tensor.astype(jnp.int32)
    res = _last_loc_sc(flat, pool, lens, stride, num_tokens)
    return res.astype(req_to_token.dtype)
